# baseline (speedup 1.0000x reference)
import functools

import jax
import jax.numpy as jnp
from jax import lax
from jax.experimental import pallas as pl
from jax.experimental.pallas import tpu as pltpu
from jax.experimental.pallas import tpu_sc as plsc

HASH_BUCKET_SIZE = 32768
GEO_BUCKETS = 4096
NCAT = 26
CAT_DIM = 16
REGION_EMB_DIM = 32
HIST = 200
NUM = 64
K = 64
TOPK = 8
HIDDEN = 256

_SQRT_HALF = 0.7071067811865476

_NC = 2
_NS = 16
_NW = _NC * _NS
_CHUNK = 128


def _sc_gather_call(tbl_flat, idx_fmajor, region_table, region_id):
    n_idx = idx_fmajor.shape[0]
    n_reg = region_id.shape[0]
    per_w = n_idx // _NW
    rper_w = n_reg // _NW
    rows_per_blk = _CHUNK * NCAT
    blk_per_w = per_w // rows_per_blk
    assert per_w % rows_per_blk == 0 and rper_w % _CHUNK == 0
    n_rin = rper_w // _CHUNK

    mesh = plsc.VectorSubcoreMesh(core_axis_name="c", subcore_axis_name="s")

    @functools.partial(
        pl.kernel,
        mesh=mesh,
        compiler_params=pltpu.CompilerParams(use_tc_tiling_on_sc=False),
        out_type=[
            jax.ShapeDtypeStruct((n_reg, NCAT * CAT_DIM), jnp.float32),
            jax.ShapeDtypeStruct((n_reg, REGION_EMB_DIM), jnp.float32),
        ],
        scratch_types=[
            pltpu.VMEM((per_w,), jnp.int32),
            pltpu.VMEM((rows_per_blk, CAT_DIM), jnp.float32),
            pltpu.VMEM((rper_w,), jnp.int32),
            pltpu.VMEM((rper_w, REGION_EMB_DIM), jnp.float32),
            pltpu.SemaphoreType.DMA,
            pltpu.SemaphoreType.DMA,
        ],
    )
    def sc_kernel(tbl_hbm, idx_hbm, rtbl_hbm, ridx_hbm, cat_out, reg_out,
                  idx_v, rows_v, ridx_v, rrows_v, sem_g, sem_o):
        w = lax.axis_index("s") * _NC + lax.axis_index("c")
        base = w * per_w

        pltpu.sync_copy(idx_hbm.at[pl.ds(base, per_w)], idx_v)

        def fix_cat(i, carry):
            pos = base + i * 16 + lax.iota(jnp.int32, 16)
            fld = lax.rem(lax.div(pos, jnp.int32(_CHUNK)), jnp.int32(NCAT))
            raw = idx_v[pl.ds(i * 16, 16)]
            clipped = jnp.minimum(jnp.maximum(raw, 0), HASH_BUCKET_SIZE - 1)
            idx_v[pl.ds(i * 16, 16)] = clipped + fld * HASH_BUCKET_SIZE
            return carry

        lax.fori_loop(0, per_w // 16, fix_cat, 0)

        def outer(o, carry):
            cps = []
            for c in range(NCAT):
                r0 = o * rows_per_blk + c * _CHUNK
                cps.append(pltpu.async_copy(
                    tbl_hbm.at[idx_v.at[pl.ds(r0, _CHUNK)]],
                    rows_v.at[pl.ds(c * _CHUNK, _CHUNK)], sem_g))
            for cp in cps:
                cp.wait()
            row0 = (w * blk_per_w + o) * _CHUNK
            ops = []
            for c in range(NCAT):
                ops.append(pltpu.async_copy(
                    rows_v.at[pl.ds(c * _CHUNK, _CHUNK)],
                    cat_out.at[pl.ds(row0, _CHUNK),
                               pl.ds(c * CAT_DIM, CAT_DIM)], sem_o))
            for cp in ops:
                cp.wait()
            return carry

        lax.fori_loop(0, blk_per_w, outer, 0)

        rbase = w * rper_w
        pltpu.sync_copy(ridx_hbm.at[pl.ds(rbase, rper_w)], ridx_v)

        def fix_reg(i, carry):
            raw = ridx_v[pl.ds(i * 16, 16)]
            ridx_v[pl.ds(i * 16, 16)] = jnp.minimum(
                jnp.maximum(raw, 0), GEO_BUCKETS - 1)
            return carry

        lax.fori_loop(0, rper_w // 16, fix_reg, 0)

        rcps = []
        for j in range(n_rin):
            rcps.append(pltpu.async_copy(
                rtbl_hbm.at[ridx_v.at[pl.ds(j * _CHUNK, _CHUNK)]],
                rrows_v.at[pl.ds(j * _CHUNK, _CHUNK)], sem_g))
        for cp in rcps:
            cp.wait()
        pltpu.sync_copy(rrows_v, reg_out.at[pl.ds(rbase, rper_w)])

    return sc_kernel(tbl_flat, idx_fmajor, region_table, region_id)


def _gelu(x):
    return 0.5 * x * (1.0 + lax.erf(x * _SQRT_HALF))


def _tc_pre_body(hist_ref, num_ref, w1d_ref, b1_ref, h0_ref):
    xd = jnp.concatenate([hist_ref[...], num_ref[...]], axis=1)
    h0_ref[...] = (jnp.dot(xd, w1d_ref[...],
                           preferred_element_type=jnp.float32) + b1_ref[...])


def _tc_body(h0_ref, cat_ref, reg_ref,
             w1s_ref,
             w2_ref, b2_ref, w3_ref, b3_ref, out_ref):
    xs = jnp.concatenate([cat_ref[...], reg_ref[...]], axis=1)
    h = h0_ref[...] + jnp.dot(xs, w1s_ref[...],
                              preferred_element_type=jnp.float32)
    h = _gelu(h)
    h = _gelu(jnp.dot(h, w2_ref[...], preferred_element_type=jnp.float32)
              + b2_ref[...])
    logits = (jnp.dot(h, w3_ref[...], preferred_element_type=jnp.float32)
              + b3_ref[...])

    bb = logits.shape[0]
    rows = lax.broadcasted_iota(jnp.int32, (K, K), 0)
    cols = lax.broadcasted_iota(jnp.int32, (K, K), 1)
    lt = jnp.where(rows <= cols, 1.0, 0.0).astype(jnp.float32)

    work = logits
    sel = jnp.zeros((bb, K), dtype=jnp.bool_)
    m1 = None
    for t in range(TOPK):
        m = jnp.max(work, axis=1, keepdims=True)
        if t == 0:
            m1 = m
        eq = work == m
        pc = jnp.dot(eq.astype(jnp.float32), lt,
                     preferred_element_type=jnp.float32)
        pick = jnp.logical_and(eq, pc == 1.0)
        sel = jnp.logical_or(sel, pick)
        work = jnp.where(pick, -jnp.inf, work)
    e = jnp.where(sel, jnp.exp(logits - m1), 0.0)
    out_ref[...] = e / jnp.sum(e, axis=1, keepdims=True)


def _tc_pre(hist_y, cur_num, W1d, b1, block_b=512):
    Bn = hist_y.shape[0]
    grid = (Bn // block_b,)
    row = lambda i: (i, 0)
    rep = lambda i: (0, 0)
    return pl.pallas_call(
        _tc_pre_body,
        grid=grid,
        in_specs=[
            pl.BlockSpec((block_b, HIST), row),
            pl.BlockSpec((block_b, NUM), row),
            pl.BlockSpec((HIST + NUM, HIDDEN), rep),
            pl.BlockSpec((1, HIDDEN), rep),
        ],
        out_specs=pl.BlockSpec((block_b, HIDDEN), row),
        out_shape=jax.ShapeDtypeStruct((Bn, HIDDEN), jnp.float32),
    )(hist_y, cur_num, W1d, b1.reshape(1, HIDDEN))


def _tc_forward(h0, cat_vec, reg_vec, W1s, W2, b2, W3, b3, block_b=512):
    Bn = h0.shape[0]
    grid = (Bn // block_b,)
    row = lambda i: (i, 0)
    rep = lambda i: (0, 0)
    return pl.pallas_call(
        _tc_body,
        grid=grid,
        in_specs=[
            pl.BlockSpec((block_b, HIDDEN), row),
            pl.BlockSpec((block_b, NCAT * CAT_DIM), row),
            pl.BlockSpec((block_b, REGION_EMB_DIM), row),
            pl.BlockSpec((NCAT * CAT_DIM + REGION_EMB_DIM, HIDDEN), rep),
            pl.BlockSpec((HIDDEN, HIDDEN), rep),
            pl.BlockSpec((1, HIDDEN), rep),
            pl.BlockSpec((HIDDEN, K), rep),
            pl.BlockSpec((1, K), rep),
        ],
        out_specs=pl.BlockSpec((block_b, K), row),
        out_shape=jax.ShapeDtypeStruct((Bn, K), jnp.float32),
    )(h0, cat_vec, reg_vec, W1s,
      W2, b2.reshape(1, HIDDEN), W3, b3.reshape(1, K))


def kernel(hist_y, cur_num, cur_cat, region_id, cat_tables, region_table,
           W1, b1, W2, b2, W3, b3):
    Bn = hist_y.shape[0]
    tbl_flat = cat_tables.reshape(NCAT * HASH_BUCKET_SIZE, CAT_DIM)
    idx_fmajor = (cur_cat.astype(jnp.int32)
                  .reshape(Bn // _CHUNK, _CHUNK, NCAT)
                  .transpose(0, 2, 1)
                  .reshape(Bn * NCAT))
    rid = region_id.astype(jnp.int32)
    cat_vec, reg_rows = _sc_gather_call(tbl_flat, idx_fmajor,
                                        region_table, rid)
    h0 = _tc_pre(hist_y, cur_num, W1[:HIST + NUM], b1)
    return _tc_forward(h0, cat_vec, reg_rows, W1[HIST + NUM:],
                       W2, b2, W3, b3)


# speedup vs baseline: 1.0263x; 1.0263x over previous
import functools

import jax
import jax.numpy as jnp
from jax import lax
from jax.experimental import pallas as pl
from jax.experimental.pallas import tpu as pltpu
from jax.experimental.pallas import tpu_sc as plsc

HASH_BUCKET_SIZE = 32768
GEO_BUCKETS = 4096
NCAT = 26
CAT_DIM = 16
REGION_EMB_DIM = 32
HIST = 200
NUM = 64
K = 64
TOPK = 8
HIDDEN = 256

_SQRT_HALF = 0.7071067811865476

_NC = 2
_NS = 16
_NW = _NC * _NS
_CHUNK = 128


def _sc_gather_call(tbl_flat, idx_flat, region_table, region_id):
    n_idx = idx_flat.shape[0]
    n_reg = region_id.shape[0]
    per_w = n_idx // _NW
    rper_w = n_reg // _NW
    rows_per_blk = _CHUNK * NCAT
    blk_per_w = per_w // rows_per_blk
    assert per_w % rows_per_blk == 0 and rper_w % _CHUNK == 0
    n_rin = rper_w // _CHUNK

    mesh = plsc.VectorSubcoreMesh(core_axis_name="c", subcore_axis_name="s")

    @functools.partial(
        pl.kernel,
        mesh=mesh,
        compiler_params=pltpu.CompilerParams(use_tc_tiling_on_sc=False),
        out_type=[
            jax.ShapeDtypeStruct((n_idx, CAT_DIM), jnp.float32),
            jax.ShapeDtypeStruct((n_reg, REGION_EMB_DIM), jnp.float32),
        ],
        scratch_types=[
            pltpu.VMEM((per_w,), jnp.int32),
            pltpu.VMEM((rows_per_blk, CAT_DIM), jnp.float32),
            pltpu.VMEM((rper_w,), jnp.int32),
            pltpu.VMEM((rper_w, REGION_EMB_DIM), jnp.float32),
            pltpu.SemaphoreType.DMA,
            pltpu.SemaphoreType.DMA,
        ],
    )
    def sc_kernel(tbl_hbm, idx_hbm, rtbl_hbm, ridx_hbm, cat_out, reg_out,
                  idx_v, rows_v, ridx_v, rrows_v, sem_g, sem_o):
        w = lax.axis_index("s") * _NC + lax.axis_index("c")
        base = w * per_w

        pltpu.sync_copy(idx_hbm.at[pl.ds(base, per_w)], idx_v)

        def fix_cat(i, carry):
            pos = base + i * 16 + lax.iota(jnp.int32, 16)
            fld = lax.rem(pos, jnp.int32(NCAT))
            raw = idx_v[pl.ds(i * 16, 16)]
            clipped = jnp.minimum(jnp.maximum(raw, 0), HASH_BUCKET_SIZE - 1)
            idx_v[pl.ds(i * 16, 16)] = clipped + fld * HASH_BUCKET_SIZE
            return carry

        lax.fori_loop(0, per_w // 16, fix_cat, 0)

        def outer(o, carry):
            cps = []
            for c in range(NCAT):
                r0 = o * rows_per_blk + c * _CHUNK
                cps.append(pltpu.async_copy(
                    tbl_hbm.at[idx_v.at[pl.ds(r0, _CHUNK)]],
                    rows_v.at[pl.ds(c * _CHUNK, _CHUNK)], sem_g))
            for cp in cps:
                cp.wait()
            pltpu.sync_copy(
                rows_v, cat_out.at[pl.ds(base + o * rows_per_blk,
                                         rows_per_blk)])
            return carry

        lax.fori_loop(0, blk_per_w, outer, 0)

        rbase = w * rper_w
        pltpu.sync_copy(ridx_hbm.at[pl.ds(rbase, rper_w)], ridx_v)

        def fix_reg(i, carry):
            raw = ridx_v[pl.ds(i * 16, 16)]
            ridx_v[pl.ds(i * 16, 16)] = jnp.minimum(
                jnp.maximum(raw, 0), GEO_BUCKETS - 1)
            return carry

        lax.fori_loop(0, rper_w // 16, fix_reg, 0)

        rcps = []
        for j in range(n_rin):
            rcps.append(pltpu.async_copy(
                rtbl_hbm.at[ridx_v.at[pl.ds(j * _CHUNK, _CHUNK)]],
                rrows_v.at[pl.ds(j * _CHUNK, _CHUNK)], sem_g))
        for cp in rcps:
            cp.wait()
        pltpu.sync_copy(rrows_v, reg_out.at[pl.ds(rbase, rper_w)])

    return sc_kernel(tbl_flat, idx_flat, region_table, region_id)


def _gelu(x):
    return 0.5 * x * (1.0 + lax.erf(x * _SQRT_HALF))


def _tc_body(hist_ref, num_ref, cat_ref, reg_ref,
             w1_ref, b1_ref,
             w2_ref, b2_ref, w3_ref, b3_ref, out_ref):
    x = jnp.concatenate([hist_ref[...], num_ref[...], cat_ref[...],
                         reg_ref[...]], axis=1)
    h = jnp.dot(x, w1_ref[...], preferred_element_type=jnp.float32) + b1_ref[...]
    h = _gelu(h)
    h = _gelu(jnp.dot(h, w2_ref[...], preferred_element_type=jnp.float32)
              + b2_ref[...])
    logits = (jnp.dot(h, w3_ref[...], preferred_element_type=jnp.float32)
              + b3_ref[...])

    bb = logits.shape[0]
    rows = lax.broadcasted_iota(jnp.int32, (K, K), 0)
    cols = lax.broadcasted_iota(jnp.int32, (K, K), 1)
    lt = jnp.where(rows <= cols, 1.0, 0.0).astype(jnp.float32)

    work = logits
    sel = jnp.zeros((bb, K), dtype=jnp.bool_)
    m1 = None
    for t in range(TOPK):
        m = jnp.max(work, axis=1, keepdims=True)
        if t == 0:
            m1 = m
        eq = work == m
        pc = jnp.dot(eq.astype(jnp.float32), lt,
                     preferred_element_type=jnp.float32)
        pick = jnp.logical_and(eq, pc == 1.0)
        sel = jnp.logical_or(sel, pick)
        work = jnp.where(pick, -jnp.inf, work)
    e = jnp.where(sel, jnp.exp(logits - m1), 0.0)
    out_ref[...] = e / jnp.sum(e, axis=1, keepdims=True)


def _tc_forward(hist_y, cur_num, cat_vec, reg_vec, W1, b1, W2, b2, W3, b3,
                block_b=512):
    Bn = hist_y.shape[0]
    grid = (Bn // block_b,)
    row = lambda i: (i, 0)
    rep = lambda i: (0, 0)
    return pl.pallas_call(
        _tc_body,
        grid=grid,
        in_specs=[
            pl.BlockSpec((block_b, HIST), row),
            pl.BlockSpec((block_b, NUM), row),
            pl.BlockSpec((block_b, NCAT * CAT_DIM), row),
            pl.BlockSpec((block_b, REGION_EMB_DIM), row),
            pl.BlockSpec((HIST + NUM + NCAT * CAT_DIM + REGION_EMB_DIM,
                          HIDDEN), rep),
            pl.BlockSpec((1, HIDDEN), rep),
            pl.BlockSpec((HIDDEN, HIDDEN), rep),
            pl.BlockSpec((1, HIDDEN), rep),
            pl.BlockSpec((HIDDEN, K), rep),
            pl.BlockSpec((1, K), rep),
        ],
        out_specs=pl.BlockSpec((block_b, K), row),
        out_shape=jax.ShapeDtypeStruct((Bn, K), jnp.float32),
    )(hist_y, cur_num, cat_vec, reg_vec,
      W1, b1.reshape(1, HIDDEN),
      W2, b2.reshape(1, HIDDEN), W3, b3.reshape(1, K))


def kernel(hist_y, cur_num, cur_cat, region_id, cat_tables, region_table,
           W1, b1, W2, b2, W3, b3):
    Bn = hist_y.shape[0]
    tbl_flat = cat_tables.reshape(NCAT * HASH_BUCKET_SIZE, CAT_DIM)
    idx_flat = cur_cat.astype(jnp.int32).reshape(Bn * NCAT)
    rid = region_id.astype(jnp.int32)
    cat_rows, reg_rows = _sc_gather_call(tbl_flat, idx_flat,
                                         region_table, rid)
    return _tc_forward(hist_y, cur_num,
                       cat_rows.reshape(Bn, NCAT * CAT_DIM), reg_rows,
                       W1, b1, W2, b2, W3, b3)


# speedup vs baseline: 1.0752x; 1.0477x over previous
import functools

import jax
import jax.numpy as jnp
from jax import lax
from jax.experimental import pallas as pl
from jax.experimental.pallas import tpu as pltpu
from jax.experimental.pallas import tpu_sc as plsc

HASH_BUCKET_SIZE = 32768
GEO_BUCKETS = 4096
NCAT = 26
CAT_DIM = 16
REGION_EMB_DIM = 32
HIST = 200
NUM = 64
K = 64
TOPK = 8
HIDDEN = 256

_SQRT_HALF = 0.7071067811865476

_NC = 2
_NS = 16
_NW = _NC * _NS
_CHUNK = 128


def _sc_gather_call(tbl_flat, idx_flat, region_table, region_id):
    n_idx = idx_flat.shape[0]
    n_reg = region_id.shape[0]
    per_w = n_idx // _NW
    rper_w = n_reg // _NW
    rows_per_blk = _CHUNK * NCAT
    blk_per_w = per_w // rows_per_blk
    assert per_w % rows_per_blk == 0 and rper_w % _CHUNK == 0
    n_rin = rper_w // _CHUNK

    mesh = plsc.VectorSubcoreMesh(core_axis_name="c", subcore_axis_name="s")

    @functools.partial(
        pl.kernel,
        mesh=mesh,
        compiler_params=pltpu.CompilerParams(use_tc_tiling_on_sc=False),
        out_type=[
            jax.ShapeDtypeStruct((n_idx, CAT_DIM), jnp.float32),
            jax.ShapeDtypeStruct((n_reg, REGION_EMB_DIM), jnp.float32),
        ],
        scratch_types=[
            pltpu.VMEM((per_w,), jnp.int32),
            pltpu.VMEM((rows_per_blk, CAT_DIM), jnp.float32),
            pltpu.VMEM((rper_w,), jnp.int32),
            pltpu.VMEM((rper_w, REGION_EMB_DIM), jnp.float32),
            pltpu.SemaphoreType.DMA,
            pltpu.SemaphoreType.DMA,
        ],
    )
    def sc_kernel(tbl_hbm, idx_hbm, rtbl_hbm, ridx_hbm, cat_out, reg_out,
                  idx_v, rows_v, ridx_v, rrows_v, sem_g, sem_o):
        w = lax.axis_index("s") * _NC + lax.axis_index("c")
        base = w * per_w

        pltpu.sync_copy(idx_hbm.at[pl.ds(base, per_w)], idx_v)

        def fix_cat(i, carry):
            pos = base + i * 16 + lax.iota(jnp.int32, 16)
            fld = lax.rem(pos, jnp.int32(NCAT))
            raw = idx_v[pl.ds(i * 16, 16)]
            clipped = jnp.minimum(jnp.maximum(raw, 0), HASH_BUCKET_SIZE - 1)
            idx_v[pl.ds(i * 16, 16)] = clipped + fld * HASH_BUCKET_SIZE
            return carry

        lax.fori_loop(0, per_w // 16, fix_cat, 0)

        def outer(o, carry):
            cps = []
            for c in range(NCAT):
                r0 = o * rows_per_blk + c * _CHUNK
                cps.append(pltpu.async_copy(
                    tbl_hbm.at[idx_v.at[pl.ds(r0, _CHUNK)]],
                    rows_v.at[pl.ds(c * _CHUNK, _CHUNK)], sem_g))
            for cp in cps:
                cp.wait()
            pltpu.sync_copy(
                rows_v, cat_out.at[pl.ds(base + o * rows_per_blk,
                                         rows_per_blk)])
            return carry

        lax.fori_loop(0, blk_per_w, outer, 0)

        rbase = w * rper_w
        pltpu.sync_copy(ridx_hbm.at[pl.ds(rbase, rper_w)], ridx_v)

        def fix_reg(i, carry):
            raw = ridx_v[pl.ds(i * 16, 16)]
            ridx_v[pl.ds(i * 16, 16)] = jnp.minimum(
                jnp.maximum(raw, 0), GEO_BUCKETS - 1)
            return carry

        lax.fori_loop(0, rper_w // 16, fix_reg, 0)

        rcps = []
        for j in range(n_rin):
            rcps.append(pltpu.async_copy(
                rtbl_hbm.at[ridx_v.at[pl.ds(j * _CHUNK, _CHUNK)]],
                rrows_v.at[pl.ds(j * _CHUNK, _CHUNK)], sem_g))
        for cp in rcps:
            cp.wait()
        pltpu.sync_copy(rrows_v, reg_out.at[pl.ds(rbase, rper_w)])

    return sc_kernel(tbl_flat, idx_flat, region_table, region_id)


def _gelu(x):
    return 0.5 * x * (1.0 + lax.erf(x * _SQRT_HALF))


def _tc_body(hist_ref, num_ref, cat_ref, reg_ref,
             w1_ref, b1_ref,
             w2_ref, b2_ref, w3_ref, b3_ref, out_ref):
    x = jnp.concatenate([hist_ref[...], num_ref[...], cat_ref[...],
                         reg_ref[...]], axis=1)
    h = jnp.dot(x, w1_ref[...], preferred_element_type=jnp.float32) + b1_ref[...]
    h = _gelu(h)
    h = _gelu(jnp.dot(h, w2_ref[...], preferred_element_type=jnp.float32)
              + b2_ref[...])
    logits = (jnp.dot(h, w3_ref[...], preferred_element_type=jnp.float32)
              + b3_ref[...])

    bb = logits.shape[0]
    rows = lax.broadcasted_iota(jnp.int32, (K, K), 0)
    cols = lax.broadcasted_iota(jnp.int32, (K, K), 1)
    lt = jnp.where(rows <= cols, 1.0, 0.0).astype(jnp.float32)

    work = logits
    sel = jnp.zeros((bb, K), dtype=jnp.bool_)
    m1 = None
    for t in range(TOPK):
        m = jnp.max(work, axis=1, keepdims=True)
        if t == 0:
            m1 = m
        eq = work == m
        pc = jnp.dot(eq.astype(jnp.float32), lt,
                     preferred_element_type=jnp.float32)
        pick = jnp.logical_and(eq, pc == 1.0)
        sel = jnp.logical_or(sel, pick)
        work = jnp.where(pick, -jnp.inf, work)
    e = jnp.where(sel, jnp.exp(logits - m1), 0.0)
    out_ref[...] = e / jnp.sum(e, axis=1, keepdims=True)


def _tc_forward(hist_y, cur_num, cat_vec, reg_vec, W1, b1, W2, b2, W3, b3,
                block_b=1024):
    Bn = hist_y.shape[0]
    grid = (Bn // block_b,)
    row = lambda i: (i, 0)
    rep = lambda i: (0, 0)
    return pl.pallas_call(
        _tc_body,
        grid=grid,
        in_specs=[
            pl.BlockSpec((block_b, HIST), row),
            pl.BlockSpec((block_b, NUM), row),
            pl.BlockSpec((block_b, NCAT * CAT_DIM), row),
            pl.BlockSpec((block_b, REGION_EMB_DIM), row),
            pl.BlockSpec((HIST + NUM + NCAT * CAT_DIM + REGION_EMB_DIM,
                          HIDDEN), rep),
            pl.BlockSpec((1, HIDDEN), rep),
            pl.BlockSpec((HIDDEN, HIDDEN), rep),
            pl.BlockSpec((1, HIDDEN), rep),
            pl.BlockSpec((HIDDEN, K), rep),
            pl.BlockSpec((1, K), rep),
        ],
        out_specs=pl.BlockSpec((block_b, K), row),
        out_shape=jax.ShapeDtypeStruct((Bn, K), jnp.float32),
    )(hist_y, cur_num, cat_vec, reg_vec,
      W1, b1.reshape(1, HIDDEN),
      W2, b2.reshape(1, HIDDEN), W3, b3.reshape(1, K))


def kernel(hist_y, cur_num, cur_cat, region_id, cat_tables, region_table,
           W1, b1, W2, b2, W3, b3):
    Bn = hist_y.shape[0]
    tbl_flat = cat_tables.reshape(NCAT * HASH_BUCKET_SIZE, CAT_DIM)
    idx_flat = cur_cat.astype(jnp.int32).reshape(Bn * NCAT)
    rid = region_id.astype(jnp.int32)
    cat_rows, reg_rows = _sc_gather_call(tbl_flat, idx_flat,
                                         region_table, rid)
    return _tc_forward(hist_y, cur_num,
                       cat_rows.reshape(Bn, NCAT * CAT_DIM), reg_rows,
                       W1, b1, W2, b2, W3, b3)


# speedup vs baseline: 1.0834x; 1.0076x over previous
import functools

import jax
import jax.numpy as jnp
from jax import lax
from jax.experimental import pallas as pl
from jax.experimental.pallas import tpu as pltpu
from jax.experimental.pallas import tpu_sc as plsc

HASH_BUCKET_SIZE = 32768
GEO_BUCKETS = 4096
NCAT = 26
CAT_DIM = 16
REGION_EMB_DIM = 32
HIST = 200
NUM = 64
K = 64
TOPK = 8
HIDDEN = 256

_SQRT_HALF = 0.7071067811865476

_NC = 2
_NS = 16
_NW = _NC * _NS
_CHUNK = 128


def _sc_gather_call(tbl_flat, idx_flat, region_table, region_id):
    n_idx = idx_flat.shape[0]
    n_reg = region_id.shape[0]
    per_w = n_idx // _NW
    rper_w = n_reg // _NW
    rows_per_blk = _CHUNK * NCAT
    blk_per_w = per_w // rows_per_blk
    assert per_w % rows_per_blk == 0 and rper_w % _CHUNK == 0
    n_rin = rper_w // _CHUNK

    mesh = plsc.VectorSubcoreMesh(core_axis_name="c", subcore_axis_name="s")

    @functools.partial(
        pl.kernel,
        mesh=mesh,
        compiler_params=pltpu.CompilerParams(use_tc_tiling_on_sc=False),
        out_type=[
            jax.ShapeDtypeStruct((n_idx, CAT_DIM), jnp.float32),
            jax.ShapeDtypeStruct((n_reg, REGION_EMB_DIM), jnp.float32),
        ],
        scratch_types=[
            pltpu.VMEM((per_w,), jnp.int32),
            pltpu.VMEM((rows_per_blk, CAT_DIM), jnp.float32),
            pltpu.VMEM((rper_w,), jnp.int32),
            pltpu.VMEM((rper_w, REGION_EMB_DIM), jnp.float32),
            pltpu.SemaphoreType.DMA,
            pltpu.SemaphoreType.DMA,
        ],
    )
    def sc_kernel(tbl_hbm, idx_hbm, rtbl_hbm, ridx_hbm, cat_out, reg_out,
                  idx_v, rows_v, ridx_v, rrows_v, sem_g, sem_o):
        w = lax.axis_index("s") * _NC + lax.axis_index("c")
        base = w * per_w

        pltpu.sync_copy(idx_hbm.at[pl.ds(base, per_w)], idx_v)

        def fix_cat(i, carry):
            pos = base + i * 16 + lax.iota(jnp.int32, 16)
            fld = lax.rem(pos, jnp.int32(NCAT))
            raw = idx_v[pl.ds(i * 16, 16)]
            clipped = jnp.minimum(jnp.maximum(raw, 0), HASH_BUCKET_SIZE - 1)
            idx_v[pl.ds(i * 16, 16)] = clipped + fld * HASH_BUCKET_SIZE
            return carry

        lax.fori_loop(0, per_w // 16, fix_cat, 0)

        def outer(o, carry):
            cps = []
            for c in range(NCAT):
                r0 = o * rows_per_blk + c * _CHUNK
                cps.append(pltpu.async_copy(
                    tbl_hbm.at[idx_v.at[pl.ds(r0, _CHUNK)]],
                    rows_v.at[pl.ds(c * _CHUNK, _CHUNK)], sem_g))
            for cp in cps:
                cp.wait()
            pltpu.sync_copy(
                rows_v, cat_out.at[pl.ds(base + o * rows_per_blk,
                                         rows_per_blk)])
            return carry

        lax.fori_loop(0, blk_per_w, outer, 0)

        rbase = w * rper_w
        pltpu.sync_copy(ridx_hbm.at[pl.ds(rbase, rper_w)], ridx_v)

        def fix_reg(i, carry):
            raw = ridx_v[pl.ds(i * 16, 16)]
            ridx_v[pl.ds(i * 16, 16)] = jnp.minimum(
                jnp.maximum(raw, 0), GEO_BUCKETS - 1)
            return carry

        lax.fori_loop(0, rper_w // 16, fix_reg, 0)

        rcps = []
        for j in range(n_rin):
            rcps.append(pltpu.async_copy(
                rtbl_hbm.at[ridx_v.at[pl.ds(j * _CHUNK, _CHUNK)]],
                rrows_v.at[pl.ds(j * _CHUNK, _CHUNK)], sem_g))
        for cp in rcps:
            cp.wait()
        pltpu.sync_copy(rrows_v, reg_out.at[pl.ds(rbase, rper_w)])

    return sc_kernel(tbl_flat, idx_flat, region_table, region_id)


def _gelu(x):
    return 0.5 * x * (1.0 + lax.erf(x * _SQRT_HALF))


def _tc_body(hist_ref, num_ref, cat_ref, reg_ref,
             w1_ref, b1_ref,
             w2_ref, b2_ref, w3_ref, b3_ref, out_ref):
    x = jnp.concatenate([hist_ref[...], num_ref[...], cat_ref[...],
                         reg_ref[...]], axis=1)
    h = jnp.dot(x, w1_ref[...], preferred_element_type=jnp.float32) + b1_ref[...]
    h = _gelu(h)
    h = _gelu(jnp.dot(h, w2_ref[...], preferred_element_type=jnp.float32)
              + b2_ref[...])
    logits = (jnp.dot(h, w3_ref[...], preferred_element_type=jnp.float32)
              + b3_ref[...])

    bb = logits.shape[0]
    rows = lax.broadcasted_iota(jnp.int32, (K, K), 0)
    cols = lax.broadcasted_iota(jnp.int32, (K, K), 1)
    lt = jnp.where(rows <= cols, 1.0, 0.0).astype(jnp.float32)

    work = logits
    sel = jnp.zeros((bb, K), dtype=jnp.bool_)
    m1 = None
    for t in range(TOPK):
        m = jnp.max(work, axis=1, keepdims=True)
        if t == 0:
            m1 = m
        eq = work == m
        pc = jnp.dot(eq.astype(jnp.float32), lt,
                     preferred_element_type=jnp.float32)
        pick = jnp.logical_and(eq, pc == 1.0)
        sel = jnp.logical_or(sel, pick)
        work = jnp.where(pick, -jnp.inf, work)
    e = jnp.where(sel, jnp.exp(logits - m1), 0.0)
    out_ref[...] = e / jnp.sum(e, axis=1, keepdims=True)


def _tc_forward(hist_y, cur_num, cat_vec, reg_vec, W1, b1, W2, b2, W3, b3,
                block_b=2048):
    Bn = hist_y.shape[0]
    grid = (Bn // block_b,)
    row = lambda i: (i, 0)
    rep = lambda i: (0, 0)
    return pl.pallas_call(
        _tc_body,
        grid=grid,
        in_specs=[
            pl.BlockSpec((block_b, HIST), row),
            pl.BlockSpec((block_b, NUM), row),
            pl.BlockSpec((block_b, NCAT * CAT_DIM), row),
            pl.BlockSpec((block_b, REGION_EMB_DIM), row),
            pl.BlockSpec((HIST + NUM + NCAT * CAT_DIM + REGION_EMB_DIM,
                          HIDDEN), rep),
            pl.BlockSpec((1, HIDDEN), rep),
            pl.BlockSpec((HIDDEN, HIDDEN), rep),
            pl.BlockSpec((1, HIDDEN), rep),
            pl.BlockSpec((HIDDEN, K), rep),
            pl.BlockSpec((1, K), rep),
        ],
        out_specs=pl.BlockSpec((block_b, K), row),
        out_shape=jax.ShapeDtypeStruct((Bn, K), jnp.float32),
    )(hist_y, cur_num, cat_vec, reg_vec,
      W1, b1.reshape(1, HIDDEN),
      W2, b2.reshape(1, HIDDEN), W3, b3.reshape(1, K))


def kernel(hist_y, cur_num, cur_cat, region_id, cat_tables, region_table,
           W1, b1, W2, b2, W3, b3):
    Bn = hist_y.shape[0]
    tbl_flat = cat_tables.reshape(NCAT * HASH_BUCKET_SIZE, CAT_DIM)
    idx_flat = cur_cat.astype(jnp.int32).reshape(Bn * NCAT)
    rid = region_id.astype(jnp.int32)
    cat_rows, reg_rows = _sc_gather_call(tbl_flat, idx_flat,
                                         region_table, rid)
    return _tc_forward(hist_y, cur_num,
                       cat_rows.reshape(Bn, NCAT * CAT_DIM), reg_rows,
                       W1, b1, W2, b2, W3, b3)
